# Initial kernel scaffold; baseline (speedup 1.0000x reference)
import jax
import jax.numpy as jnp
from jax.experimental import pallas as pl


def kernel(x, edge_index, edge_attr, W1, root1, b1, W2, root2, b2):
    raise NotImplementedError("write your pallas kernel here")


# speedup vs baseline: 1.9054x; 1.9054x over previous
import functools

import jax
import jax.numpy as jnp
from jax import lax
from jax.experimental import pallas as pl
from jax.experimental.pallas import tpu as pltpu
from jax.experimental.pallas import tpu_sc as plsc

_CH = 64
_NT = 16
_L = 16


def _make_edge_kernel(N, E, D, with_cnt):
    n_chunks = E // _CH
    assert n_chunks * _CH == E and D == 128
    stripe = (N // _NT) & ~7
    tail = N - stripe * _NT
    assert 0 <= tail <= _CH and stripe % 8 == 0 and stripe >= _CH

    mesh = plsc.VectorSubcoreMesh(core_axis_name="c", subcore_axis_name="s")

    out_type = [jax.ShapeDtypeStruct((N, D), jnp.float32),
                jax.ShapeDtypeStruct((N, D), jnp.float32)]
    if with_cnt:
        out_type.append(jax.ShapeDtypeStruct((N,), jnp.float32))

    scratch = [
        pltpu.VMEM((_CH,), jnp.int32),
        pltpu.VMEM((_CH,), jnp.int32),
        pltpu.VMEM((_CH,), jnp.float32),
        pltpu.VMEM((_CH, D), jnp.float32),
        pltpu.VMEM((_CH, D), jnp.float32),
        pltpu.VMEM((_CH, D), jnp.float32),
        pltpu.VMEM((_CH,), jnp.float32),
        pltpu.VMEM((stripe,), jnp.float32),
        pltpu.VMEM_SHARED((N, D), jnp.float32),
        pltpu.VMEM_SHARED((N,), jnp.float32),
        pltpu.SemaphoreType.DMA,
    ]

    @functools.partial(pl.kernel, out_type=out_type, mesh=mesh,
                       scratch_types=scratch)
    def edge_kernel(x_hbm, src_hbm, dst_hbm, frac_hbm, *refs):
        if with_cnt:
            a_out, b_out, cnt_out = refs[:3]
            scratch_refs = refs[3:]
        else:
            a_out, b_out = refs[:2]
            scratch_refs = refs[2:]
        (srcv, dstv, fracv, rows, scaled, zbuf, onesv, zcnt,
         acc, acccnt, sem) = scratch_refs

        cid = lax.axis_index("c")
        sid = lax.axis_index("s")

        zero16 = jnp.zeros((_L,), jnp.float32)
        one16 = jnp.ones((_L,), jnp.float32)

        def zfill(j, _):
            for t in range(D // _L):
                zbuf[j, pl.ds(t * _L, _L)] = zero16
            return 0
        lax.fori_loop(0, _CH, zfill, 0)
        for t in range(_CH // _L):
            onesv[pl.ds(t * _L, _L)] = one16
        for t in range(stripe // _L):
            zcnt[pl.ds(t * _L, _L)] = zero16

        base = sid * stripe
        n_full = stripe // _CH
        rem = stripe - n_full * _CH
        for k in range(n_full):
            pltpu.sync_copy(zbuf, acc.at[pl.ds(base + k * _CH, _CH)])
        if rem:
            pltpu.sync_copy(zbuf.at[pl.ds(0, rem)],
                            acc.at[pl.ds(base + n_full * _CH, rem)])
        if tail:
            @pl.when(sid == _NT - 1)
            def _():
                pltpu.sync_copy(zbuf.at[pl.ds(0, tail)],
                                acc.at[pl.ds(stripe * _NT, tail)])
        if with_cnt:
            @pl.when(cid == 0)
            def _():
                pltpu.sync_copy(zcnt, acccnt.at[pl.ds(base, stripe)])
                if tail:
                    @pl.when(sid == _NT - 1)
                    def _():
                        pltpu.sync_copy(zcnt.at[pl.ds(0, tail)],
                                        acccnt.at[pl.ds(stripe * _NT, tail)])
        plsc.subcore_barrier()

        n_my = (n_chunks - sid + _NT - 1) // _NT

        def chunk_body(i, _):
            cbase = (sid + i * _NT) * _CH
            pltpu.sync_copy(src_hbm.at[pl.ds(cbase, _CH)], srcv)
            pltpu.sync_copy(dst_hbm.at[pl.ds(cbase, _CH)], dstv)
            pltpu.async_copy(x_hbm.at[srcv], rows, sem).wait()

            @pl.when(cid == 0)
            def _():
                pltpu.sync_copy(rows, acc.at[dstv], add=True)
                if with_cnt:
                    pltpu.sync_copy(onesv, acccnt.at[dstv], add=True)

            @pl.when(cid == 1)
            def _():
                pltpu.sync_copy(frac_hbm.at[pl.ds(cbase, _CH)], fracv)

                def sbody(g, _):
                    f16 = fracv[pl.ds(g * _L, _L)]
                    for jj in range(_L):
                        j = g * _L + jj
                        f = f16[jj]
                        for t in range(D // _L):
                            sl = pl.ds(t * _L, _L)
                            scaled[j, sl] = rows[j, sl] * f
                    return 0
                lax.fori_loop(0, _CH // _L, sbody, 0)
                pltpu.sync_copy(scaled, acc.at[dstv], add=True)
            return 0
        lax.fori_loop(0, n_my, chunk_body, 0)

        plsc.subcore_barrier()

        def dump(src_ref, out):
            def blk(off, nrow):
                pltpu.sync_copy(src_ref.at[pl.ds(off, nrow)],
                                zbuf.at[pl.ds(0, nrow)])
                pltpu.sync_copy(zbuf.at[pl.ds(0, nrow)],
                                out.at[pl.ds(off, nrow)])
            for k in range(n_full):
                blk(base + k * _CH, _CH)
            if rem:
                blk(base + n_full * _CH, rem)
            if tail:
                @pl.when(sid == _NT - 1)
                def _():
                    blk(stripe * _NT, tail)

        @pl.when(cid == 0)
        def _():
            dump(acc, a_out)

        @pl.when(cid == 1)
        def _():
            dump(acc, b_out)

        if with_cnt:
            @pl.when(cid == 0)
            def _():
                pltpu.sync_copy(acccnt.at[pl.ds(base, stripe)], zcnt)
                pltpu.sync_copy(zcnt, cnt_out.at[pl.ds(base, stripe)])
                if tail:
                    @pl.when(sid == _NT - 1)
                    def _():
                        pltpu.sync_copy(acccnt.at[pl.ds(stripe * _NT, tail)],
                                        zcnt.at[pl.ds(0, tail)])
                        pltpu.sync_copy(zcnt.at[pl.ds(0, tail)],
                                        cnt_out.at[pl.ds(stripe * _NT, tail)])

    return edge_kernel


def _dense_body(a, b, xv, cv, w0, dw, rt, bs, o, *, act):
    f32 = jnp.float32
    acc = jnp.dot(a[...], w0[...], preferred_element_type=f32)
    acc += jnp.dot(b[...], dw[...], preferred_element_type=f32)
    acc = acc / jnp.maximum(cv[...], 1.0)
    acc = acc + jnp.dot(xv[...], rt[...], preferred_element_type=f32) + bs[...]
    if act == "relu":
        o[...] = jnp.maximum(acc, 0.0)
    else:
        m = jnp.max(acc, axis=1, keepdims=True)
        l = acc - m
        o[...] = l - jnp.log(jnp.sum(jnp.exp(l), axis=1, keepdims=True))


def _dense(a, b, x, cnt2d, w0, dw, root, bias2d, act):
    N, D = x.shape
    BN = 2000
    grid = (N // BN,)
    row_spec = lambda shp: pl.BlockSpec(shp, lambda i: (i, 0))
    w_spec = pl.BlockSpec((D, D), lambda i: (0, 0))
    return pl.pallas_call(
        functools.partial(_dense_body, act=act),
        grid=grid,
        in_specs=[row_spec((BN, D)), row_spec((BN, D)), row_spec((BN, D)),
                  row_spec((BN, 1)),
                  w_spec, w_spec, w_spec,
                  pl.BlockSpec((1, D), lambda i: (0, 0))],
        out_specs=row_spec((BN, D)),
        out_shape=jax.ShapeDtypeStruct((N, D), jnp.float32),
    )(a, b, x, cnt2d, w0, dw, root, bias2d)


def kernel(x, edge_index, edge_attr, W1, root1, b1, W2, root2, b2):
    N, D = x.shape
    E = edge_index.shape[1]
    src = edge_index[0]
    dst = edge_index[1]
    frac = edge_attr[:, 0]

    ek1 = _make_edge_kernel(N, E, D, with_cnt=True)
    ek2 = _make_edge_kernel(N, E, D, with_cnt=False)

    a, b, cnt = ek1(x, src, dst, frac)
    cnt2d = cnt.reshape(N, 1)
    h = _dense(a, b, x, cnt2d,
               W1[0], W1[1] - W1[0], root1, b1.reshape(1, D), "relu")
    a, b = ek2(h, src, dst, frac)
    return _dense(a, b, h, cnt2d,
                  W2[0], W2[1] - W2[0], root2, b2.reshape(1, D), "logsoftmax")


# speedup vs baseline: 2.1869x; 1.1477x over previous
import functools

import jax
import jax.numpy as jnp
from jax import lax
from jax.experimental import pallas as pl
from jax.experimental.pallas import tpu as pltpu
from jax.experimental.pallas import tpu_sc as plsc

_CH = 128
_NT = 16
_L = 16
_S = 4
_GC = 80
_MG = 2
_CPT = _GC * _MG


def _make_edge_kernel(N, EP, D, with_cnt):
    n_chunks = EP // _CH
    assert n_chunks == _CPT * _NT and D == 128
    NA = N + 8
    stripe = (NA // _NT) & ~7
    tail_a = NA - stripe * _NT
    tail_o = N - stripe * _NT
    assert 0 <= tail_a <= _CH and 0 <= tail_o <= _CH and stripe >= _CH

    mesh = plsc.VectorSubcoreMesh(core_axis_name="c", subcore_axis_name="s")

    out_type = [jax.ShapeDtypeStruct((N, D), jnp.float32),
                jax.ShapeDtypeStruct((N, D), jnp.float32)]
    if with_cnt:
        out_type.append(jax.ShapeDtypeStruct((N,), jnp.float32))

    scratch = [
        pltpu.VMEM((_GC, _CH), jnp.int32),
        pltpu.VMEM((_GC, _CH), jnp.int32),
        pltpu.VMEM((_GC, _CH), jnp.float32),
        pltpu.VMEM((_CH, D), jnp.float32),
        pltpu.VMEM((_CH, D), jnp.float32),
        pltpu.VMEM((_CH, D), jnp.float32),
        pltpu.VMEM((_CH, D), jnp.float32),
        pltpu.VMEM((_CH,), jnp.int32),
        pltpu.VMEM((_CH,), jnp.int32),
        pltpu.VMEM((_CH,), jnp.int32),
        pltpu.VMEM((_CH,), jnp.int32),
        pltpu.VMEM((_CH,), jnp.int32),
        pltpu.VMEM((_CH,), jnp.float32),
        pltpu.VMEM((stripe,), jnp.float32),
        pltpu.VMEM_SHARED((NA, D), jnp.float32),
        pltpu.VMEM_SHARED((NA,), jnp.float32),
    ] + [pltpu.SemaphoreType.DMA] * (2 * _S)

    @functools.partial(pl.kernel, out_type=out_type, mesh=mesh,
                       scratch_types=scratch)
    def edge_kernel(x_hbm, src_hbm, dst_hbm, frac_hbm, *refs):
        if with_cnt:
            a_out, b_out, cnt_out = refs[:3]
            srefs = refs[3:]
        else:
            a_out, b_out = refs[:2]
            srefs = refs[2:]
        (srcT, dstT, fracT, rows0, rows1, rows2, rows3,
         srcv0, srcv1, srcv2, srcv3, dstv, onesv, zcnt,
         acc, acccnt, *sems_all) = srefs
        rows = (rows0, rows1, rows2, rows3)
        srcvs = (srcv0, srcv1, srcv2, srcv3)
        semg = sems_all[:_S]
        sems = sems_all[_S:]

        cid = lax.axis_index("c")
        sid = lax.axis_index("s")

        zero16 = jnp.zeros((_L,), jnp.float32)
        one16 = jnp.ones((_L,), jnp.float32)

        def zfill(j, _):
            for t in range(D // _L):
                rows0[j, pl.ds(t * _L, _L)] = zero16
            return 0
        lax.fori_loop(0, _CH, zfill, 0)
        for t in range(_CH // _L):
            onesv[pl.ds(t * _L, _L)] = one16
        for t in range(stripe // _L):
            zcnt[pl.ds(t * _L, _L)] = zero16

        base = sid * stripe
        n_full = stripe // _CH
        rem = stripe - n_full * _CH
        for k in range(n_full):
            pltpu.sync_copy(rows0, acc.at[pl.ds(base + k * _CH, _CH)])
        if rem:
            pltpu.sync_copy(rows0.at[pl.ds(0, rem)],
                            acc.at[pl.ds(base + n_full * _CH, rem)])
        if tail_a:
            @pl.when(sid == _NT - 1)
            def _():
                pltpu.sync_copy(rows0.at[pl.ds(0, tail_a)],
                                acc.at[pl.ds(stripe * _NT, tail_a)])
        if with_cnt:
            @pl.when(cid == 0)
            def _():
                pltpu.sync_copy(zcnt, acccnt.at[pl.ds(base, stripe)])
                if tail_a:
                    @pl.when(sid == _NT - 1)
                    def _():
                        pltpu.sync_copy(zcnt.at[pl.ds(0, tail_a)],
                                        acccnt.at[pl.ds(stripe * _NT, tail_a)])
        plsc.subcore_barrier()

        def scale_rows(rb, k):
            def gbody(g, _):
                f16 = fracT[k, pl.ds(g * _L, _L)]
                for jj in range(_L):
                    j = g * _L + jj
                    f = f16[jj]
                    for t in range(D // _L):
                        sl = pl.ds(t * _L, _L)
                        rb[j, sl] = rb[j, sl] * f
                return 0
            lax.fori_loop(0, _CH // _L, gbody, 0)

        tbase = sid * _CPT
        for g2 in range(_MG):
            gbase = tbase + g2 * _GC
            pltpu.sync_copy(src_hbm.at[pl.ds(gbase, _GC)], srcT)
            pltpu.sync_copy(dst_hbm.at[pl.ds(gbase, _GC)], dstT)

            @pl.when(cid == 1)
            def _():
                pltpu.sync_copy(frac_hbm.at[pl.ds(gbase, _GC)], fracT)

            def load_idx(dst_1d, src_2d, k):
                for t in range(_CH // _L):
                    sl = pl.ds(t * _L, _L)
                    dst_1d[sl] = src_2d[k, sl]

            def body(k, _):
                load_idx(srcvs[0], srcT, k)
                pltpu.async_copy(x_hbm.at[srcvs[0]], rows[0], semg[0]).wait()

                @pl.when(cid == 1)
                def _():
                    scale_rows(rows[0], k)

                load_idx(dstv, dstT, k)
                pltpu.sync_copy(rows[0], acc.at[dstv], add=True)
                if with_cnt:
                    @pl.when(cid == 0)
                    def _():
                        pltpu.sync_copy(onesv, acccnt.at[dstv],
                                        add=True)
                return 0
            lax.fori_loop(0, _GC, body, 0)

        plsc.subcore_barrier()

        def dump(src_ref, out):
            def blk(off, nrow):
                pltpu.sync_copy(src_ref.at[pl.ds(off, nrow)],
                                rows0.at[pl.ds(0, nrow)])
                pltpu.sync_copy(rows0.at[pl.ds(0, nrow)],
                                out.at[pl.ds(off, nrow)])
            for k in range(n_full):
                blk(base + k * _CH, _CH)
            if rem:
                blk(base + n_full * _CH, rem)
            if tail_o:
                @pl.when(sid == _NT - 1)
                def _():
                    blk(stripe * _NT, tail_o)

        @pl.when(cid == 0)
        def _():
            dump(acc, a_out)

        @pl.when(cid == 1)
        def _():
            dump(acc, b_out)

        if with_cnt:
            @pl.when(cid == 0)
            def _():
                pltpu.sync_copy(acccnt.at[pl.ds(base, stripe)], zcnt)
                pltpu.sync_copy(zcnt, cnt_out.at[pl.ds(base, stripe)])
                if tail_o:
                    @pl.when(sid == _NT - 1)
                    def _():
                        pltpu.sync_copy(acccnt.at[pl.ds(stripe * _NT, tail_o)],
                                        zcnt.at[pl.ds(0, tail_o)])
                        pltpu.sync_copy(zcnt.at[pl.ds(0, tail_o)],
                                        cnt_out.at[pl.ds(stripe * _NT, tail_o)])

    return edge_kernel


def _dense_body(a, b, xv, cv, w0, dw, rt, bs, o, *, act):
    f32 = jnp.float32
    acc = jnp.dot(a[...], w0[...], preferred_element_type=f32)
    acc += jnp.dot(b[...], dw[...], preferred_element_type=f32)
    acc = acc / jnp.maximum(cv[...], 1.0)
    acc = acc + jnp.dot(xv[...], rt[...], preferred_element_type=f32) + bs[...]
    if act == "relu":
        o[...] = jnp.maximum(acc, 0.0)
    else:
        m = jnp.max(acc, axis=1, keepdims=True)
        l = acc - m
        o[...] = l - jnp.log(jnp.sum(jnp.exp(l), axis=1, keepdims=True))


def _dense(a, b, x, cnt2d, w0, dw, root, bias2d, act):
    N, D = x.shape
    BN = 2000
    grid = (N // BN,)
    row_spec = lambda shp: pl.BlockSpec(shp, lambda i: (i, 0))
    w_spec = pl.BlockSpec((D, D), lambda i: (0, 0))
    return pl.pallas_call(
        functools.partial(_dense_body, act=act),
        grid=grid,
        in_specs=[row_spec((BN, D)), row_spec((BN, D)), row_spec((BN, D)),
                  row_spec((BN, 1)),
                  w_spec, w_spec, w_spec,
                  pl.BlockSpec((1, D), lambda i: (0, 0))],
        out_specs=row_spec((BN, D)),
        out_shape=jax.ShapeDtypeStruct((N, D), jnp.float32),
    )(a, b, x, cnt2d, w0, dw, root, bias2d)


def kernel(x, edge_index, edge_attr, W1, root1, b1, W2, root2, b2):
    N, D = x.shape
    E = edge_index.shape[1]
    src = edge_index[0]
    dst = edge_index[1]
    frac = edge_attr[:, 0]

    EP = _CPT * _NT * _CH
    pad = EP - E
    assert pad >= 0
    if pad:
        src = jnp.concatenate([src, jnp.zeros((pad,), jnp.int32)])
        dst = jnp.concatenate([dst, jnp.full((pad,), N, jnp.int32)])
        frac = jnp.concatenate([frac, jnp.zeros((pad,), jnp.float32)])
    src2d = src.reshape(-1, _CH)
    dst2d = dst.reshape(-1, _CH)
    frac2d = frac.reshape(-1, _CH)

    ek1 = _make_edge_kernel(N, EP, D, with_cnt=True)

    a, b, cnt = ek1(x, src2d, dst2d, frac2d)
    cnt2d = cnt.reshape(N, 1)
    h = _dense(a, b, x, cnt2d,
               W1[0], W1[1] - W1[0], root1, b1.reshape(1, D), "relu")
    a, b, _ = ek1(h, src2d, dst2d, frac2d)
    return _dense(a, b, h, cnt2d,
                  W2[0], W2[1] - W2[0], root2, b2.reshape(1, D), "logsoftmax")


# speedup vs baseline: 2.4367x; 1.1142x over previous
import functools

import jax
import jax.numpy as jnp
from jax import lax
from jax.experimental import pallas as pl
from jax.experimental.pallas import tpu as pltpu
from jax.experimental.pallas import tpu_sc as plsc

_CH = 128
_NT = 16
_L = 16
_G = 16
_CPT = 160


def _make_edge_kernel(N, EP, D, with_cnt):
    n_chunks = EP // _CH
    assert n_chunks == _CPT * _NT and D == 128
    NA = N + 8
    stripe = (NA // _NT) & ~7
    tail_a = NA - stripe * _NT
    tail_o = N - stripe * _NT
    assert 0 <= tail_a <= _CH and 0 <= tail_o <= _CH and stripe >= _CH

    mesh = plsc.VectorSubcoreMesh(core_axis_name="c", subcore_axis_name="s")

    out_type = [jax.ShapeDtypeStruct((N, D), jnp.float32),
                jax.ShapeDtypeStruct((N, D), jnp.float32)]
    if with_cnt:
        out_type.append(jax.ShapeDtypeStruct((N,), jnp.float32))

    scratch = [
        pltpu.VMEM((_G, _CH), jnp.int32),
        pltpu.VMEM((_G, _CH), jnp.int32),
        pltpu.VMEM((_G, _CH), jnp.float32),
        pltpu.VMEM((_CH, D), jnp.float32),
        pltpu.VMEM((_CH, D), jnp.float32),
        pltpu.VMEM((_CH,), jnp.int32),
        pltpu.VMEM((_CH,), jnp.int32),
        pltpu.VMEM((_CH,), jnp.int32),
        pltpu.VMEM((_CH,), jnp.float32),
        pltpu.VMEM((stripe,), jnp.float32),
        pltpu.VMEM_SHARED((NA, D), jnp.float32),
        pltpu.VMEM_SHARED((NA,), jnp.float32),
    ] + [pltpu.SemaphoreType.DMA] * 2

    @functools.partial(pl.kernel, out_type=out_type, mesh=mesh,
                       scratch_types=scratch)
    def edge_kernel(x_hbm, src_hbm, dst_hbm, frac_hbm, *refs):
        if with_cnt:
            a_out, b_out, cnt_out = refs[:3]
            srefs = refs[3:]
        else:
            a_out, b_out = refs[:2]
            srefs = refs[2:]
        (srcT, dstT, fracT, rows0, rows1,
         srcv0, srcv1, dstv, onesv, zcnt,
         acc, acccnt, *sems_all) = srefs
        rows = (rows0, rows1)
        srcvs = (srcv0, srcv1)
        semg = sems_all

        cid = lax.axis_index("c")
        sid = lax.axis_index("s")

        zero16 = jnp.zeros((_L,), jnp.float32)
        one16 = jnp.ones((_L,), jnp.float32)

        def zfill(j, _):
            for t in range(D // _L):
                rows0[j, pl.ds(t * _L, _L)] = zero16
            return 0
        lax.fori_loop(0, _CH, zfill, 0)
        for t in range(_CH // _L):
            onesv[pl.ds(t * _L, _L)] = one16
        for t in range(stripe // _L):
            zcnt[pl.ds(t * _L, _L)] = zero16

        base = sid * stripe
        n_full = stripe // _CH
        rem = stripe - n_full * _CH
        for k in range(n_full):
            pltpu.sync_copy(rows0, acc.at[pl.ds(base + k * _CH, _CH)])
        if rem:
            pltpu.sync_copy(rows0.at[pl.ds(0, rem)],
                            acc.at[pl.ds(base + n_full * _CH, rem)])
        if tail_a:
            @pl.when(sid == _NT - 1)
            def _():
                pltpu.sync_copy(rows0.at[pl.ds(0, tail_a)],
                                acc.at[pl.ds(stripe * _NT, tail_a)])
        if with_cnt:
            @pl.when(cid == 0)
            def _():
                pltpu.sync_copy(zcnt, acccnt.at[pl.ds(base, stripe)])
                if tail_a:
                    @pl.when(sid == _NT - 1)
                    def _():
                        pltpu.sync_copy(zcnt.at[pl.ds(0, tail_a)],
                                        acccnt.at[pl.ds(stripe * _NT, tail_a)])
        plsc.subcore_barrier()

        def scale_rows(rb, k):
            def gbody(g, _):
                f16 = fracT[k, pl.ds(g * _L, _L)]
                for jj in range(_L):
                    j = g * _L + jj
                    f = f16[jj]
                    for t in range(D // _L):
                        sl = pl.ds(t * _L, _L)
                        rb[j, sl] = rb[j, sl] * f
                return 0
            lax.fori_loop(0, _CH // _L, gbody, 0)

        tbase = sid * _CPT

        def load_idx(dst_1d, src_2d, k):
            for t in range(_CH // _L):
                sl = pl.ds(t * _L, _L)
                dst_1d[sl] = src_2d[k, sl]

        def gbody(g, _):
            gbase = tbase + g * _G
            pltpu.sync_copy(src_hbm.at[pl.ds(gbase, _G)], srcT)
            pltpu.sync_copy(dst_hbm.at[pl.ds(gbase, _G)], dstT)

            @pl.when(cid == 1)
            def _():
                pltpu.sync_copy(frac_hbm.at[pl.ds(gbase, _G)], fracT)

            load_idx(srcvs[0], srcT, 0)
            pltpu.async_copy(x_hbm.at[srcvs[0]], rows[0], semg[0])
            for k in range(_G):
                b = k % 2
                if k + 1 < _G:
                    load_idx(srcvs[1 - b], srcT, k + 1)
                    pltpu.async_copy(x_hbm.at[srcvs[1 - b]], rows[1 - b],
                                     semg[1 - b])
                pltpu.make_async_copy(x_hbm.at[srcvs[b]], rows[b],
                                      semg[b]).wait()

                @pl.when(cid == 1)
                def _():
                    scale_rows(rows[b], k)

                load_idx(dstv, dstT, k)
                pltpu.sync_copy(rows[b], acc.at[dstv], add=True)
                if with_cnt:
                    @pl.when(cid == 0)
                    def _():
                        pltpu.sync_copy(onesv, acccnt.at[dstv], add=True)
            return 0
        lax.fori_loop(0, _CPT // _G, gbody, 0)

        plsc.subcore_barrier()

        def dump(src_ref, out):
            def blk(off, nrow):
                pltpu.sync_copy(src_ref.at[pl.ds(off, nrow)],
                                rows0.at[pl.ds(0, nrow)])
                pltpu.sync_copy(rows0.at[pl.ds(0, nrow)],
                                out.at[pl.ds(off, nrow)])
            for k in range(n_full):
                blk(base + k * _CH, _CH)
            if rem:
                blk(base + n_full * _CH, rem)
            if tail_o:
                @pl.when(sid == _NT - 1)
                def _():
                    blk(stripe * _NT, tail_o)

        @pl.when(cid == 0)
        def _():
            dump(acc, a_out)

        @pl.when(cid == 1)
        def _():
            dump(acc, b_out)

        if with_cnt:
            @pl.when(cid == 0)
            def _():
                pltpu.sync_copy(acccnt.at[pl.ds(base, stripe)], zcnt)
                pltpu.sync_copy(zcnt, cnt_out.at[pl.ds(base, stripe)])
                if tail_o:
                    @pl.when(sid == _NT - 1)
                    def _():
                        pltpu.sync_copy(acccnt.at[pl.ds(stripe * _NT, tail_o)],
                                        zcnt.at[pl.ds(0, tail_o)])
                        pltpu.sync_copy(zcnt.at[pl.ds(0, tail_o)],
                                        cnt_out.at[pl.ds(stripe * _NT, tail_o)])

    return edge_kernel


def _dense_body(a, b, xv, cv, w0, dw, rt, bs, o, *, act):
    f32 = jnp.float32
    acc = jnp.dot(a[...], w0[...], preferred_element_type=f32)
    acc += jnp.dot(b[...], dw[...], preferred_element_type=f32)
    acc = acc / jnp.maximum(cv[...], 1.0)
    acc = acc + jnp.dot(xv[...], rt[...], preferred_element_type=f32) + bs[...]
    if act == "relu":
        o[...] = jnp.maximum(acc, 0.0)
    else:
        m = jnp.max(acc, axis=1, keepdims=True)
        l = acc - m
        o[...] = l - jnp.log(jnp.sum(jnp.exp(l), axis=1, keepdims=True))


def _dense(a, b, x, cnt2d, w0, dw, root, bias2d, act):
    N, D = x.shape
    BN = 2000
    grid = (N // BN,)
    row_spec = lambda shp: pl.BlockSpec(shp, lambda i: (i, 0))
    w_spec = pl.BlockSpec((D, D), lambda i: (0, 0))
    return pl.pallas_call(
        functools.partial(_dense_body, act=act),
        grid=grid,
        in_specs=[row_spec((BN, D)), row_spec((BN, D)), row_spec((BN, D)),
                  row_spec((BN, 1)),
                  w_spec, w_spec, w_spec,
                  pl.BlockSpec((1, D), lambda i: (0, 0))],
        out_specs=row_spec((BN, D)),
        out_shape=jax.ShapeDtypeStruct((N, D), jnp.float32),
    )(a, b, x, cnt2d, w0, dw, root, bias2d)


def kernel(x, edge_index, edge_attr, W1, root1, b1, W2, root2, b2):
    N, D = x.shape
    E = edge_index.shape[1]
    src = edge_index[0]
    dst = edge_index[1]
    frac = edge_attr[:, 0]

    EP = _CPT * _NT * _CH
    pad = EP - E
    assert pad >= 0
    if pad:
        src = jnp.concatenate([src, jnp.zeros((pad,), jnp.int32)])
        dst = jnp.concatenate([dst, jnp.full((pad,), N, jnp.int32)])
        frac = jnp.concatenate([frac, jnp.zeros((pad,), jnp.float32)])
    src2d = src.reshape(-1, _CH)
    dst2d = dst.reshape(-1, _CH)
    frac2d = frac.reshape(-1, _CH)

    ek1 = _make_edge_kernel(N, EP, D, with_cnt=True)

    a, b, cnt = ek1(x, src2d, dst2d, frac2d)
    cnt2d = cnt.reshape(N, 1)
    h = _dense(a, b, x, cnt2d,
               W1[0], W1[1] - W1[0], root1, b1.reshape(1, D), "relu")
    a, b, _ = ek1(h, src2d, dst2d, frac2d)
    return _dense(a, b, h, cnt2d,
                  W2[0], W2[1] - W2[0], root2, b2.reshape(1, D), "logsoftmax")
